# baseline (speedup 1.0000x reference)
import functools

import jax
import jax.numpy as jnp
from jax import lax
from jax.experimental import pallas as pl
from jax.experimental.pallas import tpu as pltpu
from jax.experimental.pallas import tpu_sc as plsc

CHUNK = 512
NBUF = 2
TB = 2048


def _tc_transpose(wt):
    d, n = wt.shape

    def body(in_ref, out_ref):
        out_ref[...] = in_ref[...].T

    return pl.pallas_call(
        body,
        grid=((n + TB - 1) // TB,),
        in_specs=[pl.BlockSpec((d, TB), lambda i: (0, i))],
        out_specs=pl.BlockSpec((TB, d), lambda i: (i, 0)),
        out_shape=jax.ShapeDtypeStruct((n, d), jnp.float32),
    )(wt)


def _make_sc_gather(n_chunks, nw, nc, d):
    b_per_w = n_chunks * CHUNK
    n_outer = n_chunks // NBUF
    mesh = plsc.VectorSubcoreMesh(core_axis_name="c", subcore_axis_name="s")

    @functools.partial(
        pl.kernel,
        mesh=mesh,
        out_type=jax.ShapeDtypeStruct((nw * b_per_w, d), jnp.float32),
        scratch_types=[
            pltpu.VMEM((b_per_w,), jnp.int32),
            *[pltpu.VMEM((CHUNK, d), jnp.float32) for _ in range(NBUF)],
            *[pltpu.SemaphoreType.DMA for _ in range(2 * NBUF)],
        ],
        compiler_params=pltpu.CompilerParams(use_tc_tiling_on_sc=False),
    )
    def k(idx_hbm, table_hbm, out_hbm, idx_v, *bufs_and_sems):
        rows = bufs_and_sems[:NBUF]
        gsem = bufs_and_sems[NBUF : 2 * NBUF]
        osem = bufs_and_sems[2 * NBUF : 3 * NBUF]
        wid = lax.axis_index("s") * nc + lax.axis_index("c")
        base = wid * b_per_w
        pltpu.sync_copy(idx_hbm.at[wid], idx_v)

        def gather(j, b):
            return pltpu.async_copy(
                table_hbm.at[idx_v.at[pl.ds(j * CHUNK, CHUNK)]], rows[b], gsem[b]
            )

        def out_copy(j, b):
            return pltpu.async_copy(
                rows[b], out_hbm.at[pl.ds(base + j * CHUNK, CHUNK)], osem[b]
            )

        for b in range(NBUF):
            gather(b, b)

        def body(p, carry):
            outs = []
            for b in range(NBUF):
                j = p * NBUF + b
                pltpu.make_async_copy(
                    table_hbm.at[idx_v.at[pl.ds(j * CHUNK, CHUNK)]], rows[b], gsem[b]
                ).wait()
                outs.append(out_copy(j, b))
            for b in range(NBUF):
                outs[b].wait()

                @pl.when(p + 1 < n_outer)
                def _(b=b, p=p):
                    gather((p + 1) * NBUF + b, b)

            return carry

        lax.fori_loop(0, n_outer, body, 0)

    return k


def kernel(input_ids, weight):
    b, t = input_ids.shape
    _, d = weight.shape
    info = plsc.get_sparse_core_info()
    nc, ns = info.num_cores, info.num_subcores
    nw = nc * ns
    total = b * t
    n_chunks = total // (nw * CHUNK)
    ids = input_ids.reshape(nw, n_chunks * CHUNK).astype(jnp.int32)
    w_rm = _tc_transpose(weight.T)
    out = _make_sc_gather(n_chunks, nw, nc, d)(ids, w_rm)
    return out.reshape(b, t, d)


# speedup vs baseline: 1.0850x; 1.0850x over previous
import functools

import jax
import jax.numpy as jnp
from jax import lax
from jax.experimental import pallas as pl
from jax.experimental.pallas import tpu as pltpu
from jax.experimental.pallas import tpu_sc as plsc

CHUNK = 512
NBUF = 2
TB = 4096


def _tc_transpose(wt):
    d, n = wt.shape

    def body(in_ref, out_ref):
        x = in_ref[...]
        eye = (
            lax.broadcasted_iota(jnp.int32, (d, d), 0)
            == lax.broadcasted_iota(jnp.int32, (d, d), 1)
        ).astype(jnp.float32)
        out_ref[...] = lax.dot_general(
            x, eye, (((0,), (0,)), ((), ())), preferred_element_type=jnp.float32
        )

    return pl.pallas_call(
        body,
        grid=((n + TB - 1) // TB,),
        in_specs=[pl.BlockSpec((d, TB), lambda i: (0, i))],
        out_specs=pl.BlockSpec((TB, d), lambda i: (i, 0)),
        out_shape=jax.ShapeDtypeStruct((n, d), jnp.float32),
    )(wt)


def _make_sc_gather(n_chunks, nw, nc, d):
    b_per_w = n_chunks * CHUNK
    n_outer = n_chunks // NBUF
    mesh = plsc.VectorSubcoreMesh(core_axis_name="c", subcore_axis_name="s")

    @functools.partial(
        pl.kernel,
        mesh=mesh,
        out_type=jax.ShapeDtypeStruct((nw * b_per_w, d), jnp.float32),
        scratch_types=[
            pltpu.VMEM((b_per_w,), jnp.int32),
            *[pltpu.VMEM((CHUNK, d), jnp.float32) for _ in range(NBUF)],
            *[pltpu.SemaphoreType.DMA for _ in range(2 * NBUF)],
        ],
        compiler_params=pltpu.CompilerParams(use_tc_tiling_on_sc=False),
    )
    def k(idx_hbm, table_hbm, out_hbm, idx_v, *bufs_and_sems):
        rows = bufs_and_sems[:NBUF]
        gsem = bufs_and_sems[NBUF : 2 * NBUF]
        osem = bufs_and_sems[2 * NBUF : 3 * NBUF]
        wid = lax.axis_index("s") * nc + lax.axis_index("c")
        base = wid * b_per_w
        pltpu.sync_copy(idx_hbm.at[wid], idx_v)

        def gather(j, b):
            return pltpu.async_copy(
                table_hbm.at[idx_v.at[pl.ds(j * CHUNK, CHUNK)]], rows[b], gsem[b]
            )

        def out_copy(j, b):
            return pltpu.async_copy(
                rows[b], out_hbm.at[pl.ds(base + j * CHUNK, CHUNK)], osem[b]
            )

        for b in range(NBUF):
            gather(b, b)

        def body(p, carry):
            outs = []
            for b in range(NBUF):
                j = p * NBUF + b
                pltpu.make_async_copy(
                    table_hbm.at[idx_v.at[pl.ds(j * CHUNK, CHUNK)]], rows[b], gsem[b]
                ).wait()
                outs.append(out_copy(j, b))
            for b in range(NBUF):
                outs[b].wait()

                @pl.when(p + 1 < n_outer)
                def _(b=b, p=p):
                    gather((p + 1) * NBUF + b, b)

            return carry

        lax.fori_loop(0, n_outer, body, 0)

    return k


def kernel(input_ids, weight):
    b, t = input_ids.shape
    _, d = weight.shape
    info = plsc.get_sparse_core_info()
    nc, ns = info.num_cores, info.num_subcores
    nw = nc * ns
    total = b * t
    n_chunks = total // (nw * CHUNK)
    ids = input_ids.reshape(nw, n_chunks * CHUNK).astype(jnp.int32)
    w_rm = _tc_transpose(weight.T)
    out = _make_sc_gather(n_chunks, nw, nc, d)(ids, w_rm)
    return out.reshape(b, t, d)


# speedup vs baseline: 1.2103x; 1.1154x over previous
import functools

import jax
import jax.numpy as jnp
from jax import lax
from jax.experimental import pallas as pl
from jax.experimental.pallas import tpu as pltpu
from jax.experimental.pallas import tpu_sc as plsc

CHUNK = 512
NBUF = 2


def _make_sc_gather(n_chunks, nw, nc, d):
    b_per_w = n_chunks * CHUNK
    n_outer = n_chunks // NBUF
    mesh = plsc.VectorSubcoreMesh(core_axis_name="c", subcore_axis_name="s")

    @functools.partial(
        pl.kernel,
        mesh=mesh,
        out_type=jax.ShapeDtypeStruct((nw * b_per_w, d), jnp.float32),
        scratch_types=[
            pltpu.VMEM((b_per_w,), jnp.int32),
            *[pltpu.VMEM((CHUNK, d), jnp.float32) for _ in range(NBUF)],
            *[pltpu.SemaphoreType.DMA for _ in range(2 * NBUF)],
        ],
        compiler_params=pltpu.CompilerParams(use_tc_tiling_on_sc=False),
    )
    def k(idx_hbm, table_hbm, out_hbm, idx_v, *bufs_and_sems):
        rows = bufs_and_sems[:NBUF]
        gsem = bufs_and_sems[NBUF : 2 * NBUF]
        osem = bufs_and_sems[2 * NBUF : 3 * NBUF]
        wid = lax.axis_index("s") * nc + lax.axis_index("c")
        base = wid * b_per_w
        pltpu.sync_copy(idx_hbm.at[wid], idx_v)

        def gather(j, b):
            return pltpu.async_copy(
                table_hbm.at[idx_v.at[pl.ds(j * CHUNK, CHUNK)]], rows[b], gsem[b]
            )

        def out_copy(j, b):
            return pltpu.async_copy(
                rows[b], out_hbm.at[pl.ds(base + j * CHUNK, CHUNK)], osem[b]
            )

        for b in range(NBUF):
            gather(b, b)

        def body(p, carry):
            outs = []
            for b in range(NBUF):
                j = p * NBUF + b
                pltpu.make_async_copy(
                    table_hbm.at[idx_v.at[pl.ds(j * CHUNK, CHUNK)]], rows[b], gsem[b]
                ).wait()
                outs.append(out_copy(j, b))
            for b in range(NBUF):
                outs[b].wait()

                @pl.when(p + 1 < n_outer)
                def _(b=b, p=p):
                    gather((p + 1) * NBUF + b, b)

            return carry

        lax.fori_loop(0, n_outer, body, 0)

    return k


def kernel(input_ids, weight):
    b, t = input_ids.shape
    _, d = weight.shape
    info = plsc.get_sparse_core_info()
    nc, ns = info.num_cores, info.num_subcores
    nw = nc * ns
    total = b * t
    n_chunks = total // (nw * CHUNK)
    ids = input_ids.reshape(nw, n_chunks * CHUNK).astype(jnp.int32)
    out = _make_sc_gather(n_chunks, nw, nc, d)(ids, weight)
    return out.reshape(b, t, d)


# speedup vs baseline: 1.2867x; 1.0631x over previous
import functools

import jax
import jax.numpy as jnp
from jax import lax
from jax.experimental import pallas as pl
from jax.experimental.pallas import tpu as pltpu
from jax.experimental.pallas import tpu_sc as plsc

CHUNK = 512
NBUF = 2
TB = 32768
UPC = CHUNK // 128


def _tc_transpose(wt):
    d, n = wt.shape

    def body(in_ref, out_ref):
        out_ref[...] = in_ref[...].T

    return pl.pallas_call(
        body,
        grid=((n + TB - 1) // TB,),
        in_specs=[pl.BlockSpec((d, TB), lambda i: (0, i))],
        out_specs=pl.BlockSpec((TB, d), lambda i: (i, 0)),
        out_shape=jax.ShapeDtypeStruct((n, d), jnp.float32),
    )(wt)


def _make_sc_gather(n_units, nw, nc, nt, ncol):
    n_chunks = n_units // UPC
    n_outer = n_chunks // NBUF
    b_per_w = n_units * 128
    mesh = plsc.VectorSubcoreMesh(core_axis_name="c", subcore_axis_name="s")

    @functools.partial(
        pl.kernel,
        mesh=mesh,
        out_type=jax.ShapeDtypeStruct((nt, 8, ncol, 8, 128), jnp.float32),
        scratch_types=[
            pltpu.VMEM((b_per_w,), jnp.int32),
            *[pltpu.VMEM((CHUNK, 64), jnp.float32) for _ in range(NBUF)],
            *[pltpu.VMEM((8, 8, 129), jnp.float32) for _ in range(2)],
            *[pltpu.SemaphoreType.DMA for _ in range(NBUF + 2)],
        ],
        compiler_params=pltpu.CompilerParams(
            use_tc_tiling_on_sc=False, needs_layout_passes=False
        ),
    )
    def k(idx_hbm, table_hbm, out_hbm, idx_v, *refs):
        rows = refs[:NBUF]
        tbuf = refs[NBUF : NBUF + 2]
        gsem = refs[NBUF + 2 : 2 * NBUF + 2]
        osem = refs[2 * NBUF + 2 : 2 * NBUF + 4]
        wid = lax.axis_index("s") * nc + lax.axis_index("c")
        base_u = wid * n_units
        pltpu.sync_copy(idx_hbm.at[pl.ds(wid * b_per_w, b_per_w)], idx_v)

        iota = lax.iota(jnp.int32, 16)
        rs = [(16 * q + iota) >> 3 for q in range(4)]
        ss = [(16 * q + iota) & 7 for q in range(4)]

        def gather(j, b):
            return pltpu.async_copy(
                table_hbm.at[idx_v.at[pl.ds(j * CHUNK, CHUNK)]], rows[b], gsem[b]
            )

        def out_dma(t, c, tb):
            return pltpu.async_copy(
                tbuf[tb].at[:, :, pl.ds(0, 128)],
                out_hbm.at[t, :, c, :, :],
                osem[tb],
            )

        for b in range(NBUF):
            gather(b, b)

        def body(p, carry):
            for b in range(NBUF):
                j = p * NBUF + b
                pltpu.make_async_copy(
                    table_hbm.at[idx_v.at[pl.ds(j * CHUNK, CHUNK)]], rows[b], gsem[b]
                ).wait()
                for q in range(UPC):
                    u = base_u + j * UPC + q
                    t = u // ncol
                    c = lax.rem(u, ncol)
                    tb = q % 2

                    @pl.when(j * UPC + q >= 2)
                    def _(t=t, c=c, tb=tb):
                        pltpu.make_async_copy(
                            tbuf[tb].at[:, :, pl.ds(0, 128)],
                            out_hbm.at[t, :, c, :, :],
                            osem[tb],
                        ).wait()

                    def row_body(l, carry2, b=b, q=q, tb=tb):
                        l_vec = iota * 0 + l
                        row_vec = iota * 0 + (q * 128 + l)
                        for kk in range(4):
                            vals = plsc.load_gather(
                                rows[b], [row_vec, 16 * kk + iota]
                            )
                            plsc.store_scatter(
                                tbuf[tb], [rs[kk], ss[kk], l_vec], vals
                            )
                        return carry2

                    lax.fori_loop(0, 128, row_body, 0)
                    out_dma(t, c, tb)

            for b in range(NBUF):

                @pl.when(p + 1 < n_outer)
                def _(b=b, p=p):
                    gather((p + 1) * NBUF + b, b)

            return carry

        lax.fori_loop(0, n_outer, body, 0)

        for tb in range(2):
            pltpu.make_async_copy(
                tbuf[tb].at[:, :, pl.ds(0, 128)],
                out_hbm.at[0, :, 0, :, :],
                osem[tb],
            ).wait()

    return k


def kernel(input_ids, weight):
    b, t = input_ids.shape
    v, d = weight.shape
    info = plsc.get_sparse_core_info()
    nc, ns = info.num_cores, info.num_subcores
    nw = nc * ns
    ncol = b // 128
    n_units_total = t * ncol
    n_units = n_units_total // nw
    idsf = input_ids.T.reshape(-1).astype(jnp.int32)
    table_rm = _tc_transpose(weight.T)
    out5 = _make_sc_gather(n_units, nw, nc, t, ncol)(idsf, table_rm)
    return out5.transpose((2, 4, 0, 1, 3)).reshape(b, t, d)


# speedup vs baseline: 1.8064x; 1.4039x over previous
import functools

import jax
import jax.numpy as jnp
from jax import lax
from jax.experimental import pallas as pl
from jax.experimental.pallas import tpu as pltpu
from jax.experimental.pallas import tpu_sc as plsc

CHUNK = 256
NBUF = 2
TB = 32768
UPC = CHUNK // 128


def _tc_transpose(wt):
    d, n = wt.shape

    def body(in_ref, out_ref):
        x = in_ref[...].T
        out_ref[...] = jnp.concatenate(
            [x, jnp.zeros((TB, 128 - d), jnp.float32)], axis=1
        )

    return pl.pallas_call(
        body,
        grid=((n + TB - 1) // TB,),
        in_specs=[pl.BlockSpec((d, TB), lambda i: (0, i))],
        out_specs=pl.BlockSpec((TB, 128), lambda i: (i, 0)),
        out_shape=jax.ShapeDtypeStruct((n, 128), jnp.float32),
    )(wt)


def _make_sc_gather(n_units, nw, nc, nt, ncol):
    n_chunks = n_units // UPC
    n_outer = n_chunks // NBUF
    b_per_w = n_units * 128
    mesh = plsc.VectorSubcoreMesh(core_axis_name="c", subcore_axis_name="s")

    @functools.partial(
        pl.kernel,
        mesh=mesh,
        out_type=jax.ShapeDtypeStruct((nt, 8, ncol, 8, 128), jnp.float32),
        scratch_types=[
            pltpu.VMEM((b_per_w,), jnp.int32),
            *[pltpu.VMEM((CHUNK, 128), jnp.float32) for _ in range(NBUF)],
            *[pltpu.VMEM((8, 8, 129), jnp.float32) for _ in range(2)],
            *[pltpu.SemaphoreType.DMA for _ in range(NBUF + 2)],
        ],
        compiler_params=pltpu.CompilerParams(
            use_tc_tiling_on_sc=False, needs_layout_passes=False
        ),
    )
    def k(idx_hbm, table_hbm, out_hbm, idx_v, *refs):
        rows = refs[:NBUF]
        tbuf = refs[NBUF : NBUF + 2]
        gsem = refs[NBUF + 2 : 2 * NBUF + 2]
        osem = refs[2 * NBUF + 2 : 2 * NBUF + 4]
        wid = lax.axis_index("s") * nc + lax.axis_index("c")
        base_u = wid * n_units
        pltpu.sync_copy(idx_hbm.at[pl.ds(wid * b_per_w, b_per_w)], idx_v)

        iota = lax.iota(jnp.int32, 16)
        rs = [(16 * q + iota) >> 3 for q in range(4)]
        ss = [(16 * q + iota) & 7 for q in range(4)]

        def gather(j, b):
            return pltpu.async_copy(
                table_hbm.at[idx_v.at[pl.ds(j * CHUNK, CHUNK)]], rows[b], gsem[b]
            )

        def out_dma(t, c, tb):
            return pltpu.async_copy(
                tbuf[tb].at[:, :, pl.ds(0, 128)],
                out_hbm.at[t, :, c, :, :],
                osem[tb],
            )

        for b in range(NBUF):
            gather(b, b)

        def body(p, carry):
            for b in range(NBUF):
                j = p * NBUF + b
                pltpu.make_async_copy(
                    table_hbm.at[idx_v.at[pl.ds(j * CHUNK, CHUNK)]], rows[b], gsem[b]
                ).wait()
                for q in range(UPC):
                    u = base_u + j * UPC + q
                    t = u // ncol
                    c = lax.rem(u, ncol)
                    tb = q % 2

                    @pl.when(j * UPC + q >= 2)
                    def _(t=t, c=c, tb=tb):
                        pltpu.make_async_copy(
                            tbuf[tb].at[:, :, pl.ds(0, 128)],
                            out_hbm.at[t, :, c, :, :],
                            osem[tb],
                        ).wait()

                    def row_body(l, carry2, b=b, q=q, tb=tb):
                        l_vec = iota * 0 + l
                        for kk in range(4):
                            vals = rows[b][q * 128 + l, pl.ds(16 * kk, 16)]
                            plsc.store_scatter(
                                tbuf[tb], [rs[kk], ss[kk], l_vec], vals
                            )
                        return carry2

                    lax.fori_loop(0, 128, row_body, 0)
                    out_dma(t, c, tb)

            for b in range(NBUF):

                @pl.when(p + 1 < n_outer)
                def _(b=b, p=p):
                    gather((p + 1) * NBUF + b, b)

            return carry

        lax.fori_loop(0, n_outer, body, 0)

        for tb in range(2):
            pltpu.make_async_copy(
                tbuf[tb].at[:, :, pl.ds(0, 128)],
                out_hbm.at[0, :, 0, :, :],
                osem[tb],
            ).wait()

    return k


def kernel(input_ids, weight):
    b, t = input_ids.shape
    v, d = weight.shape
    info = plsc.get_sparse_core_info()
    nc, ns = info.num_cores, info.num_subcores
    nw = nc * ns
    ncol = b // 128
    n_units_total = t * ncol
    n_units = n_units_total // nw
    idsf = input_ids.T.reshape(-1).astype(jnp.int32)
    table_pad = _tc_transpose(weight.T)
    out5 = _make_sc_gather(n_units, nw, nc, t, ncol)(idsf, table_pad)
    return out5.transpose((2, 4, 0, 1, 3)).reshape(b, t, d)


# speedup vs baseline: 1.8427x; 1.0201x over previous
import functools

import jax
import jax.numpy as jnp
from jax import lax
from jax.experimental import pallas as pl
from jax.experimental.pallas import tpu as pltpu
from jax.experimental.pallas import tpu_sc as plsc

CHUNK = 256
NBUF = 2
TB = 32768
UPC = CHUNK // 128


def _tc_transpose(wt):
    d, n = wt.shape

    def body(in_ref, out_ref):
        x = in_ref[...].T
        out_ref[...] = jnp.concatenate(
            [x, jnp.zeros((TB, 128 - d), jnp.float32)], axis=1
        )

    return pl.pallas_call(
        body,
        grid=((n + TB - 1) // TB,),
        in_specs=[pl.BlockSpec((d, TB), lambda i: (0, i))],
        out_specs=pl.BlockSpec((TB, 128), lambda i: (i, 0)),
        out_shape=jax.ShapeDtypeStruct((n, 128), jnp.float32),
    )(wt)


def _make_sc_gather(n_units, nw, nc, nt, ncol):
    n_chunks = n_units // UPC
    n_outer = n_chunks // NBUF
    b_per_w = n_units * 128
    mesh = plsc.VectorSubcoreMesh(core_axis_name="c", subcore_axis_name="s")

    @functools.partial(
        pl.kernel,
        mesh=mesh,
        out_type=jax.ShapeDtypeStruct((nt, 8, ncol, 8, 128), jnp.float32),
        scratch_types=[
            pltpu.VMEM((b_per_w,), jnp.int32),
            *[pltpu.VMEM((CHUNK, 128), jnp.float32) for _ in range(NBUF)],
            *[pltpu.VMEM((8, 8, 129), jnp.float32) for _ in range(2)],
            *[pltpu.SemaphoreType.DMA for _ in range(NBUF + 2)],
        ],
        compiler_params=pltpu.CompilerParams(
            use_tc_tiling_on_sc=False, needs_layout_passes=False
        ),
    )
    def k(idx_hbm, table_hbm, out_hbm, idx_v, *refs):
        rows = refs[:NBUF]
        tbuf = refs[NBUF : NBUF + 2]
        gsem = refs[NBUF + 2 : 2 * NBUF + 2]
        osem = refs[2 * NBUF + 2 : 2 * NBUF + 4]
        wid = lax.axis_index("s") * nc + lax.axis_index("c")
        base_u = wid * n_units
        pltpu.sync_copy(idx_hbm.at[pl.ds(wid * b_per_w, b_per_w)], idx_v)

        iota = lax.iota(jnp.int32, 16)
        rs = [(16 * q + iota) >> 3 for q in range(4)]
        ss = [(16 * q + iota) & 7 for q in range(4)]

        def gather(j, b):
            return pltpu.async_copy(
                table_hbm.at[idx_v.at[pl.ds(j * CHUNK, CHUNK)]], rows[b], gsem[b]
            )

        def out_dma(t, c, tb):
            return pltpu.async_copy(
                tbuf[tb].at[:, :, pl.ds(0, 128)],
                out_hbm.at[t, :, c, :, :],
                osem[tb],
            )

        for b in range(NBUF):
            gather(b, b)

        def body(p, carry):
            for b in range(NBUF):
                j = p * NBUF + b
                pltpu.make_async_copy(
                    table_hbm.at[idx_v.at[pl.ds(j * CHUNK, CHUNK)]], rows[b], gsem[b]
                ).wait()
                for q in range(UPC):
                    u = base_u + j * UPC + q
                    t = u // ncol
                    c = lax.rem(u, ncol)
                    tb = q % 2

                    @pl.when(j * UPC + q >= 2)
                    def _(t=t, c=c, tb=tb):
                        pltpu.make_async_copy(
                            tbuf[tb].at[:, :, pl.ds(0, 128)],
                            out_hbm.at[t, :, c, :, :],
                            osem[tb],
                        ).wait()

                    def row_body(l, carry2, b=b, q=q, tb=tb):
                        l_vec = iota * 0 + l
                        for kk in range(4):
                            vals = rows[b][q * 128 + l, pl.ds(16 * kk, 16)]
                            plsc.store_scatter(
                                tbuf[tb], [rs[kk], ss[kk], l_vec], vals
                            )
                        return carry2

                    lax.fori_loop(0, 128, row_body, 0, unroll=4)
                    out_dma(t, c, tb)

            for b in range(NBUF):

                @pl.when(p + 1 < n_outer)
                def _(b=b, p=p):
                    gather((p + 1) * NBUF + b, b)

            return carry

        lax.fori_loop(0, n_outer, body, 0)

        for tb in range(2):
            pltpu.make_async_copy(
                tbuf[tb].at[:, :, pl.ds(0, 128)],
                out_hbm.at[0, :, 0, :, :],
                osem[tb],
            ).wait()

    return k


def kernel(input_ids, weight):
    b, t = input_ids.shape
    v, d = weight.shape
    info = plsc.get_sparse_core_info()
    nc, ns = info.num_cores, info.num_subcores
    nw = nc * ns
    ncol = b // 128
    n_units_total = t * ncol
    n_units = n_units_total // nw
    idsf = input_ids.T.reshape(-1).astype(jnp.int32)
    table_pad = _tc_transpose(weight.T)
    out5 = _make_sc_gather(n_units, nw, nc, t, ncol)(idsf, table_pad)
    return out5.transpose((2, 4, 0, 1, 3)).reshape(b, t, d)


# speedup vs baseline: 1.8624x; 1.0107x over previous
import functools

import jax
import jax.numpy as jnp
from jax import lax
from jax.experimental import pallas as pl
from jax.experimental.pallas import tpu as pltpu
from jax.experimental.pallas import tpu_sc as plsc

CHUNK = 256
NBUF = 2
TB = 32768
UPC = CHUNK // 128


def _tc_transpose(wt):
    d, n = wt.shape

    def body(in_ref, out_ref):
        x = in_ref[...].T
        out_ref[...] = jnp.concatenate(
            [x, jnp.zeros((TB, 128 - d), jnp.float32)], axis=1
        )

    return pl.pallas_call(
        body,
        grid=((n + TB - 1) // TB,),
        in_specs=[pl.BlockSpec((d, TB), lambda i: (0, i))],
        out_specs=pl.BlockSpec((TB, 128), lambda i: (i, 0)),
        out_shape=jax.ShapeDtypeStruct((n, 128), jnp.float32),
    )(wt)


def _make_sc_gather(n_units, nw, nc, nt, ncol):
    n_chunks = n_units // UPC
    n_outer = n_chunks // NBUF
    b_per_w = n_units * 128
    mesh = plsc.VectorSubcoreMesh(core_axis_name="c", subcore_axis_name="s")

    @functools.partial(
        pl.kernel,
        mesh=mesh,
        out_type=jax.ShapeDtypeStruct((nt, 8, ncol, 8, 128), jnp.float32),
        scratch_types=[
            pltpu.VMEM((b_per_w,), jnp.int32),
            *[pltpu.VMEM((CHUNK, 128), jnp.float32) for _ in range(NBUF)],
            *[pltpu.VMEM((8, 8, 129), jnp.float32) for _ in range(2)],
            *[pltpu.SemaphoreType.DMA for _ in range(NBUF + 2)],
        ],
        compiler_params=pltpu.CompilerParams(
            use_tc_tiling_on_sc=False, needs_layout_passes=False
        ),
    )
    def k(idx_hbm, table_hbm, out_hbm, idx_v, *refs):
        rows = refs[:NBUF]
        tbuf = refs[NBUF : NBUF + 2]
        gsem = refs[NBUF + 2 : 2 * NBUF + 2]
        osem = refs[2 * NBUF + 2 : 2 * NBUF + 4]
        wid = lax.axis_index("s") * nc + lax.axis_index("c")
        base_u = wid * n_units
        pltpu.sync_copy(idx_hbm.at[pl.ds(wid * b_per_w, b_per_w)], idx_v)

        iota = lax.iota(jnp.int32, 16)
        rs = [(16 * q + iota) >> 3 for q in range(4)]
        ss = [(16 * q + iota) & 7 for q in range(4)]

        def gather(j, b):
            return pltpu.async_copy(
                table_hbm.at[idx_v.at[pl.ds(j * CHUNK, CHUNK)]], rows[b], gsem[b]
            )

        def out_dma(t, c, tb):
            return pltpu.async_copy(
                tbuf[tb].at[:, :, pl.ds(0, 128)],
                out_hbm.at[t, :, c, :, :],
                osem[tb],
            )

        for b in range(NBUF):
            gather(b, b)

        def body(p, carry):
            for b in range(NBUF):
                j = p * NBUF + b
                pltpu.make_async_copy(
                    table_hbm.at[idx_v.at[pl.ds(j * CHUNK, CHUNK)]], rows[b], gsem[b]
                ).wait()
                for q in range(UPC):
                    u = base_u + j * UPC + q
                    t = u // ncol
                    c = lax.rem(u, ncol)
                    tb = q % 2

                    @pl.when(j * UPC + q >= 2)
                    def _(t=t, c=c, tb=tb):
                        pltpu.make_async_copy(
                            tbuf[tb].at[:, :, pl.ds(0, 128)],
                            out_hbm.at[t, :, c, :, :],
                            osem[tb],
                        ).wait()

                    def row_body(l, l_vec, b=b, q=q, tb=tb):
                        for kk in range(4):
                            vals = rows[b][q * 128 + l, pl.ds(16 * kk, 16)]
                            plsc.store_scatter(
                                tbuf[tb], [rs[kk], ss[kk], l_vec], vals
                            )
                        return l_vec + 1

                    lax.fori_loop(0, 128, row_body, iota * 0, unroll=8)
                    out_dma(t, c, tb)

            for b in range(NBUF):

                @pl.when(p + 1 < n_outer)
                def _(b=b, p=p):
                    gather((p + 1) * NBUF + b, b)

            return carry

        lax.fori_loop(0, n_outer, body, 0)

        for tb in range(2):
            pltpu.make_async_copy(
                tbuf[tb].at[:, :, pl.ds(0, 128)],
                out_hbm.at[0, :, 0, :, :],
                osem[tb],
            ).wait()

    return k


def kernel(input_ids, weight):
    b, t = input_ids.shape
    v, d = weight.shape
    info = plsc.get_sparse_core_info()
    nc, ns = info.num_cores, info.num_subcores
    nw = nc * ns
    ncol = b // 128
    n_units_total = t * ncol
    n_units = n_units_total // nw
    idsf = input_ids.T.reshape(-1).astype(jnp.int32)
    table_pad = _tc_transpose(weight.T)
    out5 = _make_sc_gather(n_units, nw, nc, t, ncol)(idsf, table_pad)
    return out5.transpose((2, 4, 0, 1, 3)).reshape(b, t, d)
